# baseline (speedup 1.0000x reference)
import functools

import numpy as np
import jax
import jax.numpy as jnp
from jax import lax
from jax.experimental import pallas as pl
from jax.experimental.pallas import tpu as pltpu
from jax.experimental.pallas import tpu_sc as plsc

EMBED = 32
MAX_LEN = 202
SEQ = 200
BATCH = 4096
N_TOK = BATCH * SEQ
NUM_WORKERS = 32
PER_W = N_TOK // NUM_WORKERS
CHUNK = 3200
N_CHUNKS = PER_W // CHUNK
KROWS = CHUNK // 128
IDX_ROWS = N_TOK // 128
N_CHUNKS_ALL = N_TOK // CHUNK


def _make_pe() -> jnp.ndarray:
    pos = np.arange(MAX_LEN, dtype=np.float32)[:, None]
    div = np.exp(np.arange(0, EMBED, 2, dtype=np.float32) * -(np.log(10000.0) / EMBED))
    pe = np.zeros((MAX_LEN, EMBED), dtype=np.float32)
    pe[:, 0::2] = np.sin(pos * div)
    pe[:, 1::2] = np.cos(pos * div)
    return jnp.asarray(pe[:SEQ])


_PIDX = (np.arange(PER_W, dtype=np.int32) % SEQ).reshape(PER_W // 128, 128)

_MESH = plsc.VectorSubcoreMesh(core_axis_name="c", subcore_axis_name="s")


@functools.partial(
    pl.kernel,
    out_type=jax.ShapeDtypeStruct((N_TOK, EMBED), jnp.float32),
    mesh=_MESH,
    compiler_params=pltpu.CompilerParams(use_tc_tiling_on_sc=False),
    scratch_types=[
        pltpu.VMEM((5, CHUNK), jnp.int32),
        pltpu.VMEM((CHUNK, EMBED), jnp.float32),
        pltpu.SemaphoreType.DMA,
        pltpu.SemaphoreType.DMA,
        pltpu.SemaphoreType.DMA,
    ],
)
def _emb_kernel(grid_tab, time_tab, event_tab, hand_tab, pe_tab, comb_idx, out,
                s_idx, r_acc, sem_i, sem_g, sem_a):
    wid = lax.axis_index("s") * 2 + lax.axis_index("c")

    def chunk_body(i, carry):
        cid = wid * N_CHUNKS + i
        pltpu.async_copy(comb_idx.at[cid], s_idx, sem_i).wait()
        pltpu.async_copy(grid_tab.at[s_idx.at[0]], r_acc, sem_g).wait()
        acps = [
            pltpu.async_copy(time_tab.at[s_idx.at[1]], r_acc, sem_a, add=True),
            pltpu.async_copy(event_tab.at[s_idx.at[2]], r_acc, sem_a, add=True),
            pltpu.async_copy(hand_tab.at[s_idx.at[3]], r_acc, sem_a, add=True),
            pltpu.async_copy(pe_tab.at[s_idx.at[4]], r_acc, sem_a, add=True),
        ]
        for cp in acps:
            cp.wait()
        pltpu.sync_copy(r_acc, out.at[pl.ds(cid * CHUNK, CHUNK)])
        return carry

    lax.fori_loop(0, N_CHUNKS, chunk_body, 0)


def kernel(grid, timestamp, event, hand, grid_table, time_table, event_table,
           hand_table, train_mode):
    pe = _make_pe()
    gi = grid.astype(jnp.int32).reshape(IDX_ROWS, 128)
    ti = timestamp.astype(jnp.int32).reshape(IDX_ROWS, 128)
    ei = event.astype(jnp.int32).reshape(IDX_ROWS, 128)
    hi = hand.astype(jnp.int32).reshape(IDX_ROWS, 128)
    pi = jnp.asarray(np.tile(_PIDX, (NUM_WORKERS, 1)))
    comb = (jnp.stack([gi, ti, ei, hi, pi])
            .reshape(5, N_CHUNKS_ALL, CHUNK)
            .transpose(1, 0, 2))
    out = _emb_kernel(grid_table, time_table, event_table, hand_table, pe, comb)
    return out.reshape(BATCH, SEQ, EMBED)


# speedup vs baseline: 3.4999x; 3.4999x over previous
import functools

import numpy as np
import jax
import jax.numpy as jnp
from jax import lax
from jax.experimental import pallas as pl
from jax.experimental.pallas import tpu as pltpu
from jax.experimental.pallas import tpu_sc as plsc

EMBED = 32
MAX_LEN = 202
SEQ = 200
BATCH = 4096
N_TOK = BATCH * SEQ
NUM_WORKERS = 32
PER_W = N_TOK // NUM_WORKERS
CHUNK = 2 * SEQ
N_CHUNKS = PER_W // CHUNK
N_EH = 39
N_COMBO = 52 * N_EH
UNROLL = 8


def _make_pe() -> jnp.ndarray:
    pos = np.arange(MAX_LEN, dtype=np.float32)[:, None]
    div = np.exp(np.arange(0, EMBED, 2, dtype=np.float32) * -(np.log(10000.0) / EMBED))
    pe = np.zeros((MAX_LEN, EMBED), dtype=np.float32)
    pe[:, 0::2] = np.sin(pos * div)
    pe[:, 1::2] = np.cos(pos * div)
    return jnp.asarray(pe[:SEQ])


_MESH = plsc.VectorSubcoreMesh(core_axis_name="c", subcore_axis_name="s")


@functools.partial(
    pl.kernel,
    out_type=jax.ShapeDtypeStruct((N_TOK, EMBED), jnp.float32),
    mesh=_MESH,
    compiler_params=pltpu.CompilerParams(use_tc_tiling_on_sc=False,
                                         needs_layout_passes=False),
    scratch_types=[
        pltpu.VMEM((2, CHUNK), jnp.int32),
        pltpu.VMEM((2, CHUNK), jnp.int32),
        pltpu.VMEM((2, CHUNK, EMBED), jnp.float32),
        pltpu.VMEM((52, EMBED), jnp.float32),
        pltpu.VMEM((13, EMBED), jnp.float32),
        pltpu.VMEM((3, EMBED), jnp.float32),
        pltpu.VMEM((N_EH, EMBED), jnp.float32),
        pltpu.VMEM((N_COMBO, EMBED), jnp.float32),
        pltpu.VMEM((CHUNK, EMBED), jnp.float32),
        pltpu.SemaphoreType.DMA,
        pltpu.SemaphoreType.DMA,
        pltpu.SemaphoreType.DMA,
        pltpu.SemaphoreType.DMA,
    ],
)
def _emb_kernel(grid_tab, time_tab, event_tab, hand_tab, pe_tab, gidx, cidx, out,
                s_gi, s_ci, acc, time_v, event_v, hand_v, eh_v, combo_v, pe2_v,
                sem_gi, sem_ci, sem_g, sem_w):
    wid = lax.axis_index("s") * 2 + lax.axis_index("c")
    tok0 = wid * PER_W

    pltpu.sync_copy(time_tab, time_v)
    pltpu.sync_copy(event_tab, event_v)
    pltpu.sync_copy(hand_tab, hand_v)
    pltpu.sync_copy(pe_tab, pe2_v.at[pl.ds(0, SEQ)])
    pltpu.sync_copy(pe_tab, pe2_v.at[pl.ds(SEQ, SEQ)])

    def eh_body(j, carry):
        e = j // 3
        h = j - e * 3
        for c0 in (0, 16):
            eh_v[j, c0:c0 + 16] = event_v[e, c0:c0 + 16] + hand_v[h, c0:c0 + 16]
        return carry

    lax.fori_loop(0, N_EH, eh_body, 0)

    def combo_body(r, carry):
        t = r // N_EH
        j = r - t * N_EH
        for c0 in (0, 16):
            combo_v[r, c0:c0 + 16] = time_v[t, c0:c0 + 16] + eh_v[j, c0:c0 + 16]
        return carry

    lax.fori_loop(0, N_COMBO, combo_body, 0)

    def fire_idx(i):
        slot = lax.rem(i, 2)
        base = tok0 + i * CHUNK
        pltpu.async_copy(gidx.at[pl.ds(base, CHUNK)], s_gi.at[slot], sem_gi)
        pltpu.async_copy(cidx.at[pl.ds(base, CHUNK)], s_ci.at[slot], sem_ci)

    def wait_idx():
        pltpu.make_async_copy(gidx.at[pl.ds(0, CHUNK)], s_gi.at[0], sem_gi).wait()
        pltpu.make_async_copy(cidx.at[pl.ds(0, CHUNK)], s_ci.at[0], sem_ci).wait()

    def fire_gather(i):
        slot = lax.rem(i, 2)
        pltpu.async_copy(grid_tab.at[s_gi.at[slot]], acc.at[slot], sem_g)

    def wait_gather():
        pltpu.make_async_copy(grid_tab.at[s_gi.at[0]], acc.at[0], sem_g).wait()

    def fire_wb(i):
        slot = lax.rem(i, 2)
        base = tok0 + i * CHUNK
        pltpu.async_copy(acc.at[slot], out.at[pl.ds(base, CHUNK)], sem_w)

    def wait_wb():
        pltpu.make_async_copy(acc.at[0], out.at[pl.ds(0, CHUNK)], sem_w).wait()

    def chunk_body(i, carry):
        slot = lax.rem(i, 2)
        fire_idx(i)
        wait_idx()
        fire_gather(i)
        wait_gather()

        def tok_body(g, c2):
            row16 = lax.iota(jnp.int32, 16) + g * 16
            c16 = s_ci[slot, pl.ds(g * 16, 16)]
            for c in range(EMBED):
                colsplat = jnp.full((16,), c, dtype=jnp.int32)
                v = (plsc.load_gather(combo_v, [c16, colsplat])
                     + plsc.load_gather(pe2_v, [row16, colsplat]))
                plsc.addupdate_scatter(acc.at[slot], [row16, colsplat], v)
            return c2

        lax.fori_loop(0, CHUNK // 16, tok_body, 0)
        base = tok0 + i * CHUNK
        pltpu.sync_copy(acc.at[slot], out.at[pl.ds(base, CHUNK)])
        return carry

    lax.fori_loop(0, N_CHUNKS, chunk_body, 0)


def kernel(grid, timestamp, event, hand, grid_table, time_table, event_table,
           hand_table, train_mode):
    pe = _make_pe()
    gi = grid.astype(jnp.int32).reshape(N_TOK)
    ci = (timestamp.astype(jnp.int32) * N_EH + event.astype(jnp.int32) * 3
          + hand.astype(jnp.int32)).reshape(N_TOK)
    out = _emb_kernel(grid_table, time_table, event_table, hand_table, pe, gi, ci)
    return out.reshape(BATCH, SEQ, EMBED)


# speedup vs baseline: 7.2680x; 2.0766x over previous
import functools

import numpy as np
import jax
import jax.numpy as jnp
from jax import lax
from jax.experimental import pallas as pl
from jax.experimental.pallas import tpu as pltpu
from jax.experimental.pallas import tpu_sc as plsc

EMBED = 32
MAX_LEN = 202
SEQ = 200
BATCH = 4096
N_TOK = BATCH * SEQ
NUM_WORKERS = 32
PER_W = N_TOK // NUM_WORKERS
CHUNK = 2 * SEQ
N_CHUNKS = PER_W // CHUNK
N_EH = 39
N_COMBO = 52 * N_EH
UNROLL = 8


def _make_pe() -> jnp.ndarray:
    pos = np.arange(MAX_LEN, dtype=np.float32)[:, None]
    div = np.exp(np.arange(0, EMBED, 2, dtype=np.float32) * -(np.log(10000.0) / EMBED))
    pe = np.zeros((MAX_LEN, EMBED), dtype=np.float32)
    pe[:, 0::2] = np.sin(pos * div)
    pe[:, 1::2] = np.cos(pos * div)
    return jnp.asarray(pe[:SEQ])


_MESH = plsc.VectorSubcoreMesh(core_axis_name="c", subcore_axis_name="s")


@functools.partial(
    pl.kernel,
    out_type=jax.ShapeDtypeStruct((N_TOK, EMBED), jnp.float32),
    mesh=_MESH,
    compiler_params=pltpu.CompilerParams(use_tc_tiling_on_sc=False,
                                         needs_layout_passes=False),
    scratch_types=[
        pltpu.VMEM((2, CHUNK), jnp.int32),
        pltpu.VMEM((2, CHUNK), jnp.int32),
        pltpu.VMEM((2, CHUNK, EMBED), jnp.float32),
        pltpu.VMEM((52, EMBED), jnp.float32),
        pltpu.VMEM((13, EMBED), jnp.float32),
        pltpu.VMEM((3, EMBED), jnp.float32),
        pltpu.VMEM((N_EH, EMBED), jnp.float32),
        pltpu.VMEM((N_COMBO, EMBED), jnp.float32),
        pltpu.VMEM((CHUNK, EMBED), jnp.float32),
        pltpu.SemaphoreType.DMA,
        pltpu.SemaphoreType.DMA,
        pltpu.SemaphoreType.DMA,
        pltpu.SemaphoreType.DMA,
        pltpu.SemaphoreType.DMA,
        pltpu.SemaphoreType.DMA,
    ],
)
def _emb_kernel(grid_tab, time_tab, event_tab, hand_tab, pe_tab, gidx, cidx, out,
                s_gi, s_ci, acc, time_v, event_v, hand_v, eh_v, combo_v, pe2_v,
                sem_i0, sem_i1, sem_g0, sem_g1, sem_w0, sem_w1):
    sem_i = (sem_i0, sem_i1)
    sem_g = (sem_g0, sem_g1)
    sem_w = (sem_w0, sem_w1)
    wid = lax.axis_index("s") * 2 + lax.axis_index("c")
    tok0 = wid * PER_W

    pltpu.sync_copy(time_tab, time_v)
    pltpu.sync_copy(event_tab, event_v)
    pltpu.sync_copy(hand_tab, hand_v)
    pltpu.sync_copy(pe_tab, pe2_v.at[pl.ds(0, SEQ)])
    pltpu.sync_copy(pe_tab, pe2_v.at[pl.ds(SEQ, SEQ)])

    def eh_body(j, carry):
        e = j // 3
        h = j - e * 3
        for c0 in (0, 16):
            eh_v[j, c0:c0 + 16] = event_v[e, c0:c0 + 16] + hand_v[h, c0:c0 + 16]
        return carry

    lax.fori_loop(0, N_EH, eh_body, 0)

    def combo_body(r, carry):
        t = r // N_EH
        j = r - t * N_EH
        for c0 in (0, 16):
            combo_v[r, c0:c0 + 16] = time_v[t, c0:c0 + 16] + eh_v[j, c0:c0 + 16]
        return carry

    lax.fori_loop(0, N_COMBO, combo_body, 0)

    iota16 = lax.iota(jnp.int32, 16)

    def fire_idx(i, slot):
        base = tok0 + i * CHUNK
        pltpu.async_copy(gidx.at[pl.ds(base, CHUNK)], s_gi.at[slot], sem_i[slot])
        pltpu.async_copy(cidx.at[pl.ds(base, CHUNK)], s_ci.at[slot], sem_i[slot])

    def wait_idx(slot):
        pltpu.make_async_copy(gidx.at[pl.ds(0, CHUNK)], s_gi.at[slot], sem_i[slot]).wait()
        pltpu.make_async_copy(cidx.at[pl.ds(0, CHUNK)], s_ci.at[slot], sem_i[slot]).wait()

    def fire_gather(slot):
        pltpu.async_copy(grid_tab.at[s_gi.at[slot]], acc.at[slot], sem_g[slot])

    def wait_gather(slot):
        pltpu.make_async_copy(grid_tab.at[s_gi.at[slot]], acc.at[slot], sem_g[slot]).wait()

    def fire_wb(i, slot):
        base = tok0 + i * CHUNK
        pltpu.async_copy(acc.at[slot], out.at[pl.ds(base, CHUNK)], sem_w[slot])

    def wait_wb(slot):
        pltpu.make_async_copy(acc.at[slot], out.at[pl.ds(0, CHUNK)], sem_w[slot]).wait()

    def compute(slot):
        def tok_body(g, c2):
            row16 = iota16 + g * 16
            c16 = s_ci[slot, pl.ds(g * 16, 16)]
            for d in range(EMBED):
                col16 = lax.bitwise_and(iota16 + d, EMBED - 1)
                v = (plsc.load_gather(combo_v, [c16, col16])
                     + plsc.load_gather(pe2_v, [row16, col16]))
                plsc.addupdate_scatter(acc.at[slot], [row16, col16], v)
            return c2

        lax.fori_loop(0, CHUNK // 16, tok_body, 0)

    def step(i, slot):
        wait_gather(slot)
        compute(slot)
        fire_wb(i, slot)

        @pl.when(i + 2 < N_CHUNKS)
        def _():
            fire_idx(i + 2, slot)
            wait_idx(slot)
            wait_wb(slot)
            fire_gather(slot)

    fire_idx(0, 0)
    fire_idx(1, 1)
    wait_idx(0)
    fire_gather(0)
    wait_idx(1)
    fire_gather(1)

    def pair_body(pr, carry):
        step(2 * pr, 0)
        step(2 * pr + 1, 1)
        return carry

    lax.fori_loop(0, N_CHUNKS // 2, pair_body, 0)
    wait_wb(0)
    wait_wb(1)


def kernel(grid, timestamp, event, hand, grid_table, time_table, event_table,
           hand_table, train_mode):
    pe = _make_pe()
    gi = grid.astype(jnp.int32).reshape(N_TOK)
    ci = (timestamp.astype(jnp.int32) * N_EH + event.astype(jnp.int32) * 3
          + hand.astype(jnp.int32)).reshape(N_TOK)
    out = _emb_kernel(grid_table, time_table, event_table, hand_table, pe, gi, ci)
    return out.reshape(BATCH, SEQ, EMBED)
